# baseline (speedup 1.0000x reference)
import functools

import jax
import jax.numpy as jnp
from jax import lax
from jax.experimental import pallas as pl
from jax.experimental.pallas import tpu as pltpu
from jax.experimental.pallas import tpu_sc as plsc

_B = 16384
_D = 32
_CHUNK = 128


@functools.cache
def _build():
    info = plsc.get_sparse_core_info()
    nw = info.num_cores * info.num_subcores
    b_per_w = _B // nw
    n_chunks = b_per_w // _CHUNK
    mesh = plsc.VectorSubcoreMesh(core_axis_name="c", subcore_axis_name="s")

    @functools.partial(
        pl.kernel,
        mesh=mesh,
        compiler_params=pltpu.CompilerParams(use_tc_tiling_on_sc=False),
        out_type=jax.ShapeDtypeStruct((_B, _D), jnp.float32),
        scratch_types=[
            pltpu.VMEM((b_per_w,), jnp.int32),
            pltpu.VMEM((b_per_w, _D), jnp.float32),
            pltpu.SemaphoreType.DMA,
        ],
    )
    def gather_kernel(idx_hbm, table_hbm, out_hbm, idx_v, rows_v, sem):
        wid = lax.axis_index("s") * info.num_cores + lax.axis_index("c")
        base = wid * b_per_w
        pltpu.sync_copy(idx_hbm.at[pl.ds(base, b_per_w)], idx_v)
        copies = []
        for j in range(n_chunks):
            copies.append(
                pltpu.async_copy(
                    table_hbm.at[idx_v.at[pl.ds(j * _CHUNK, _CHUNK)]],
                    rows_v.at[pl.ds(j * _CHUNK, _CHUNK)],
                    sem,
                )
            )
        for c in copies:
            c.wait()
        pltpu.sync_copy(rows_v, out_hbm.at[pl.ds(base, b_per_w)])

    return gather_kernel


def kernel(beta, Emb):
    return _build()(beta.astype(jnp.int32), Emb)


# speedup vs baseline: 3.5721x; 3.5721x over previous
import functools

import jax
import jax.numpy as jnp
from jax import lax
from jax.experimental import pallas as pl
from jax.experimental.pallas import tpu as pltpu
from jax.experimental.pallas import tpu_sc as plsc

_B = 16384
_D = 32
_CHUNK = 16


@functools.cache
def _build():
    info = plsc.get_sparse_core_info()
    nw = info.num_cores * info.num_subcores
    b_per_w = _B // nw
    n_chunks = b_per_w // _CHUNK
    mesh = plsc.VectorSubcoreMesh(core_axis_name="c", subcore_axis_name="s")

    @functools.partial(
        pl.kernel,
        mesh=mesh,
        compiler_params=pltpu.CompilerParams(needs_layout_passes=False),
        out_type=jax.ShapeDtypeStruct((_D, _B), jnp.float32),
        scratch_types=[
            pltpu.VMEM((b_per_w,), jnp.int32),
            pltpu.VMEM((_CHUNK, _D, 128), jnp.float32),
            pltpu.VMEM((_D, b_per_w), jnp.float32),
            pltpu.SemaphoreType.DMA,
            pltpu.SemaphoreType.DMA,
        ],
    )
    def gather_kernel(idx_hbm, tab_hbm, out_hbm, idx_v, blk_v, rows_v, sem_i, sem):
        wid = lax.axis_index("s") * info.num_cores + lax.axis_index("c")
        base = pl.multiple_of(wid * b_per_w, 128)
        pltpu.async_copy(idx_hbm.at[pl.ds(base, b_per_w)], idx_v, sem_i).wait()
        lane = lax.iota(jnp.int32, 16)

        @pl.loop(0, n_chunks)
        def _chunk(jj):
            cv = idx_v[pl.ds(jj * _CHUNK, 16)]
            copies = []
            scal = []
            for kk in range(_CHUNK):
                c = jnp.sum(jnp.where(lane == kk, cv, 0))
                t = pl.multiple_of(c - c % 128, 128)
                scal.append(c % 128)
                copies.append(
                    pltpu.async_copy(
                        tab_hbm.at[:, pl.ds(t, 128)], blk_v.at[kk], sem
                    )
                )
            for cp in copies:
                cp.wait()
            for kk in range(_CHUNK):
                r = scal[kk]
                kcol = jnp.full((16,), kk, dtype=jnp.int32)
                rcol = jnp.full((16,), r, dtype=jnp.int32)
                jcol = jnp.full((16,), jj * _CHUNK + kk, dtype=jnp.int32)
                lo = plsc.load_gather(blk_v, [kcol, lane, rcol])
                hi = plsc.load_gather(blk_v, [kcol, lane + 16, rcol])
                plsc.store_scatter(rows_v, [lane, jcol], lo)
                plsc.store_scatter(rows_v, [lane + 16, jcol], hi)

        pltpu.sync_copy(rows_v, out_hbm.at[:, pl.ds(base, b_per_w)])

    return gather_kernel


def kernel(beta, Emb):
    out_t = _build()(beta.astype(jnp.int32), Emb.T)
    return out_t.T


# speedup vs baseline: 3.8028x; 1.0646x over previous
import functools

import jax
import jax.numpy as jnp
from jax import lax
from jax.experimental import pallas as pl
from jax.experimental.pallas import tpu as pltpu
from jax.experimental.pallas import tpu_sc as plsc

_B = 16384
_D = 32
_CHUNK = 8


@functools.cache
def _build():
    info = plsc.get_sparse_core_info()
    nw = info.num_cores * info.num_subcores
    b_per_w = _B // nw
    n_pairs = b_per_w // (2 * _CHUNK)
    mesh = plsc.VectorSubcoreMesh(core_axis_name="c", subcore_axis_name="s")

    @functools.partial(
        pl.kernel,
        mesh=mesh,
        compiler_params=pltpu.CompilerParams(needs_layout_passes=False),
        out_type=jax.ShapeDtypeStruct((_D, _B), jnp.float32),
        scratch_types=[
            pltpu.VMEM((b_per_w,), jnp.int32),
            pltpu.VMEM((2, _CHUNK, _D, 128), jnp.float32),
            pltpu.VMEM((_D, b_per_w), jnp.float32),
            pltpu.SemaphoreType.DMA,
            pltpu.SemaphoreType.DMA,
            pltpu.SemaphoreType.DMA,
        ],
    )
    def gather_kernel(
        idx_hbm, tab_hbm, out_hbm, idx_v, blk_v, rows_v, sem_i, sem_a, sem_b
    ):
        wid = lax.axis_index("s") * info.num_cores + lax.axis_index("c")
        base = pl.multiple_of(wid * b_per_w, 128)
        pltpu.async_copy(idx_hbm.at[pl.ds(base, b_per_w)], idx_v, sem_i).wait()
        lane = lax.iota(jnp.int32, 16)

        def fire(cv, half, buf, sem):
            for kk in range(_CHUNK):
                c = jnp.sum(jnp.where(lane == half * _CHUNK + kk, cv, 0))
                t = pl.multiple_of(c - c % 128, 128)
                pltpu.async_copy(
                    tab_hbm.at[:, pl.ds(t, 128)], blk_v.at[buf, kk], sem
                )

        def drain(sem):
            for kk in range(_CHUNK):
                pltpu.make_async_copy(
                    tab_hbm.at[:, pl.ds(0, 128)], blk_v.at[0, kk], sem
                ).wait()

        def extract(cv, half, buf, jbase):
            for kk in range(_CHUNK):
                c = jnp.sum(jnp.where(lane == half * _CHUNK + kk, cv, 0))
                r = c % 128
                kcol = jnp.full((16,), kk, dtype=jnp.int32)
                bcol = jnp.full((16,), buf, dtype=jnp.int32)
                rcol = jnp.full((16,), r, dtype=jnp.int32)
                jcol = jnp.full((16,), jbase + kk, dtype=jnp.int32)
                lo = plsc.load_gather(blk_v, [bcol, kcol, lane, rcol])
                hi = plsc.load_gather(blk_v, [bcol, kcol, lane + 16, rcol])
                plsc.store_scatter(rows_v, [lane, jcol], lo)
                plsc.store_scatter(rows_v, [lane + 16, jcol], hi)

        cv0 = idx_v[pl.ds(0, 16)]
        fire(cv0, 0, 0, sem_a)

        @pl.loop(0, n_pairs)
        def _pair(pp):
            cv = idx_v[pl.ds(pp * 16, 16)]
            jbase = pp * 16
            fire(cv, 1, 1, sem_b)
            drain(sem_a)
            extract(cv, 0, 0, jbase)

            @pl.when(pp < n_pairs - 1)
            def _prefetch():
                cvn = idx_v[pl.ds(pp * 16 + 16, 16)]
                fire(cvn, 0, 0, sem_a)

            drain(sem_b)
            extract(cv, 1, 1, jbase + _CHUNK)

        pltpu.sync_copy(rows_v, out_hbm.at[:, pl.ds(base, b_per_w)])

    return gather_kernel


def kernel(beta, Emb):
    out_t = _build()(beta.astype(jnp.int32), Emb.T)
    return out_t.T
